# baseline (speedup 1.0000x reference)
import functools

import jax
import jax.numpy as jnp
from jax import lax
from jax.experimental import pallas as pl
from jax.experimental.pallas import tpu as pltpu
from jax.experimental.pallas import tpu_sc as plsc

_N = 10000
_H = 64
_HC = _H // 2

_NC = 2
_NS = 16
_CHUNK = 512
_BLK = 40
_EPW = _BLK * _CHUNK
_EPAD = _NS * _EPW
_ACC_ROWS = 10112
_PER_SUB = _ACC_ROWS // _NS
_TAIL = _N - (_NS - 1) * _PER_SUB


def _bn_relu(y):
    mu = jnp.mean(y, axis=0, keepdims=True)
    yc = y - mu
    var = jnp.mean(yc * yc, axis=0, keepdims=True)
    return jnp.maximum(yc * lax.rsqrt(var + 1e-5), 0.0)


def _encoder_body(xr_ref, xp_ref, wer_ref, wep_ref, wh_ref,
                  feat_ref, fsplit_ref):
    hr = _bn_relu(jnp.dot(xr_ref[...], wer_ref[...],
                          preferred_element_type=jnp.float32))
    hp = _bn_relu(jnp.dot(xp_ref[...], wep_ref[...],
                          preferred_element_type=jnp.float32))
    f = hr + hp
    feat = _bn_relu(jnp.dot(f, wh_ref[...],
                            preferred_element_type=jnp.float32))
    feat_ref[...] = feat
    fsplit_ref[0] = feat[:, :_HC]
    fsplit_ref[1] = feat[:, _HC:]


def _head_body(feat_ref, p0_ref, p1_ref, wg_ref, wa_ref, ba_ref, wb_ref,
               bb_ref, wc_ref, wt_ref, bt_ref, out_ref):
    f = feat_ref[...]
    agg = jnp.concatenate([p0_ref[:_N, :], p1_ref[:_N, :]], axis=1)
    g = _bn_relu(jnp.dot(f + agg, wg_ref[...],
                         preferred_element_type=jnp.float32))
    a = jnp.tanh(jnp.dot(g, wa_ref[...],
                         preferred_element_type=jnp.float32) + ba_ref[...])
    b = jax.nn.sigmoid(jnp.dot(g, wb_ref[...],
                               preferred_element_type=jnp.float32)
                       + bb_ref[...])
    s = jnp.dot(a * b, wc_ref[...],
                preferred_element_type=jnp.float32)
    m = jnp.max(s)
    e = jnp.exp(s - m)
    z = jnp.sum(e)
    pooled = jnp.sum(e * g, axis=0, keepdims=True) / z
    out_ref[...] = jnp.dot(pooled, wt_ref[...],
                           preferred_element_type=jnp.float32) + bt_ref[...]


@functools.lru_cache(maxsize=1)
def _get_sc_segment_sum():
    mesh = plsc.VectorSubcoreMesh(core_axis_name="c", subcore_axis_name="s",
                                  num_cores=1)

    @functools.partial(
        pl.kernel,
        out_type=jax.ShapeDtypeStruct((_ACC_ROWS, _HC), jnp.float32),
        mesh=mesh,
        compiler_params=pltpu.CompilerParams(use_tc_tiling_on_sc=False),
        scratch_types=[
            pltpu.VMEM((_BLK, _CHUNK), jnp.int32),
            pltpu.VMEM((_BLK, _CHUNK), jnp.int32),
            pltpu.VMEM((_CHUNK, _HC), jnp.float32),
            pltpu.VMEM((_CHUNK, _HC), jnp.float32),
            pltpu.VMEM_SHARED((_ACC_ROWS, _HC), jnp.float32),
            pltpu.VMEM_SHARED((_N, _HC), jnp.float32),
            pltpu.SemaphoreType.DMA,
            pltpu.SemaphoreType.DMA,
            pltpu.SemaphoreType.DMA,
            pltpu.SemaphoreType.DMA,
        ],
    )
    def _sc_segment_sum(fhalf_hbm, src_hbm, dst_hbm, zeros_hbm, out_hbm,
                        sidx, didx, rows0, rows1, acc, feat_s,
                        gsem0, gsem1, ssem0, ssem1):
        s = lax.axis_index("s")
        pltpu.sync_copy(src_hbm.at[s], sidx)
        pltpu.sync_copy(dst_hbm.at[s], didx)
        pltpu.sync_copy(zeros_hbm, acc.at[pl.ds(s * _PER_SUB, _PER_SUB)])

        @pl.when(s < _NS - 1)
        def _():
            pltpu.sync_copy(
                fhalf_hbm.at[pl.ds(s * _PER_SUB, _PER_SUB)],
                feat_s.at[pl.ds(s * _PER_SUB, _PER_SUB)])

        @pl.when(s == _NS - 1)
        def _():
            pltpu.sync_copy(
                fhalf_hbm.at[pl.ds((_NS - 1) * _PER_SUB, _TAIL)],
                feat_s.at[pl.ds((_NS - 1) * _PER_SUB, _TAIL)])

        plsc.subcore_barrier()

        def body(i, carry):
            j0 = i * 2
            j1 = j0 + 1
            g0 = pltpu.async_copy(feat_s.at[sidx.at[j0]], rows0, gsem0)
            g1 = pltpu.async_copy(feat_s.at[sidx.at[j1]], rows1, gsem1)
            g0.wait()
            s0 = pltpu.async_copy(rows0, acc.at[didx.at[j0]], ssem0,
                                  add=True)
            g1.wait()
            s1 = pltpu.async_copy(rows1, acc.at[didx.at[j1]], ssem1,
                                  add=True)
            s0.wait()
            s1.wait()
            return carry

        lax.fori_loop(0, _BLK // 2, body, 0)
        plsc.subcore_barrier()
        pltpu.sync_copy(acc.at[pl.ds(s * _PER_SUB, _PER_SUB)],
                        out_hbm.at[pl.ds(s * _PER_SUB, _PER_SUB)])

    return _sc_segment_sum


def kernel(x_radiomics, x_pathomics, W_er, b_er, W_ep, b_ep, W_h, b_h,
           W_g, b_g, Wa, ba, Wb, bb, Wc, bc, W_t, b_t, edge_index):
    n, h = _N, _H

    feat, fsplit = pl.pallas_call(
        _encoder_body,
        out_shape=(jax.ShapeDtypeStruct((n, h), jnp.float32),
                   jax.ShapeDtypeStruct((_NC, n, _HC), jnp.float32)),
    )(x_radiomics, x_pathomics, W_er, W_ep, W_h)

    src = edge_index[0]
    dst = edge_index[1]
    e = src.shape[0]
    pad = _EPAD - e
    src_p = jnp.concatenate(
        [src, jnp.zeros((pad,), jnp.int32)]).reshape(_NS, _BLK, _CHUNK)
    dst_p = jnp.concatenate(
        [dst, jnp.full((pad,), n, jnp.int32)]).reshape(_NS, _BLK, _CHUNK)
    zeros = jnp.zeros((_PER_SUB, _HC), jnp.float32)

    sc = _get_sc_segment_sum()
    part0 = sc(fsplit[0], src_p, dst_p, zeros)
    part1 = sc(fsplit[1], src_p, dst_p, zeros)

    out = pl.pallas_call(
        _head_body,
        out_shape=jax.ShapeDtypeStruct((1, 1), jnp.float32),
    )(feat, part0, part1, W_g, Wa, ba.reshape(1, -1), Wb, bb.reshape(1, -1),
      Wc, W_t, b_t.reshape(1, 1))
    return out


# speedup vs baseline: 1.0797x; 1.0797x over previous
import functools

import jax
import jax.numpy as jnp
from jax import lax
from jax.experimental import pallas as pl
from jax.experimental.pallas import tpu as pltpu
from jax.experimental.pallas import tpu_sc as plsc

_N = 10000
_H = 64
_HC = _H // 2

_NC = 2
_NS = 16
_CHUNK = 512
_BLK = 40
_EPW = _BLK * _CHUNK
_EPAD = _NS * _EPW
_ACC_ROWS = 10112
_PER_SUB = _ACC_ROWS // _NS
_TAIL = _N - (_NS - 1) * _PER_SUB


def _bn_relu(y):
    mu = jnp.mean(y, axis=0, keepdims=True)
    yc = y - mu
    var = jnp.mean(yc * yc, axis=0, keepdims=True)
    return jnp.maximum(yc * lax.rsqrt(var + 1e-5), 0.0)


def _encoder_body(xr_ref, xp_ref, wer_ref, wep_ref, wh_ref, feat_ref):
    hr = _bn_relu(jnp.dot(xr_ref[...], wer_ref[...],
                          preferred_element_type=jnp.float32))
    hp = _bn_relu(jnp.dot(xp_ref[...], wep_ref[...],
                          preferred_element_type=jnp.float32))
    f = hr + hp
    feat_ref[...] = _bn_relu(jnp.dot(f, wh_ref[...],
                                     preferred_element_type=jnp.float32))


def _head_body(feat_ref, p0_ref, p1_ref, wg_ref, wa_ref, ba_ref, wb_ref,
               bb_ref, wc_ref, wt_ref, bt_ref, out_ref):
    f = feat_ref[...]
    agg = jnp.concatenate([p0_ref[:_N, :], p1_ref[:_N, :]], axis=1)
    g = _bn_relu(jnp.dot(f + agg, wg_ref[...],
                         preferred_element_type=jnp.float32))
    a = jnp.tanh(jnp.dot(g, wa_ref[...],
                         preferred_element_type=jnp.float32) + ba_ref[...])
    b = jax.nn.sigmoid(jnp.dot(g, wb_ref[...],
                               preferred_element_type=jnp.float32)
                       + bb_ref[...])
    s = jnp.dot(a * b, wc_ref[...],
                preferred_element_type=jnp.float32)
    m = jnp.max(s)
    e = jnp.exp(s - m)
    z = jnp.sum(e)
    pooled = jnp.sum(e * g, axis=0, keepdims=True) / z
    out_ref[...] = jnp.dot(pooled, wt_ref[...],
                           preferred_element_type=jnp.float32) + bt_ref[...]


@functools.lru_cache(maxsize=1)
def _get_sc_segment_sum():
    mesh = plsc.VectorSubcoreMesh(core_axis_name="c", subcore_axis_name="s")

    @functools.partial(
        pl.kernel,
        out_type=(jax.ShapeDtypeStruct((_ACC_ROWS, _HC), jnp.float32),
                  jax.ShapeDtypeStruct((_ACC_ROWS, _HC), jnp.float32)),
        mesh=mesh,
        compiler_params=pltpu.CompilerParams(use_tc_tiling_on_sc=False,
                                             needs_layout_passes=False),
        scratch_types=[
            pltpu.VMEM((_BLK, _CHUNK), jnp.int32),
            pltpu.VMEM((_BLK, _CHUNK), jnp.int32),
            pltpu.VMEM((_CHUNK, _HC), jnp.bfloat16),
            pltpu.VMEM((_CHUNK, _HC), jnp.bfloat16),
            pltpu.VMEM((_CHUNK, _HC), jnp.float32),
            pltpu.VMEM((_CHUNK, _HC), jnp.float32),
            pltpu.VMEM_SHARED((_ACC_ROWS, _HC), jnp.float32),
            pltpu.VMEM_SHARED((_N, _HC), jnp.bfloat16),
            pltpu.SemaphoreType.DMA,
            pltpu.SemaphoreType.DMA,
            pltpu.SemaphoreType.DMA,
            pltpu.SemaphoreType.DMA,
        ],
    )
    def _sc_segment_sum(fsplit_hbm, src_hbm, dst_hbm, zeros_hbm,
                        out0_hbm, out1_hbm,
                        sidx, didx, rb0, rb1, rows0, rows1, acc, feat_s,
                        gsem0, gsem1, ssem0, ssem1):
        c = lax.axis_index("c")
        s = lax.axis_index("s")
        pltpu.sync_copy(src_hbm.at[s], sidx)
        pltpu.sync_copy(dst_hbm.at[s], didx)
        pltpu.sync_copy(zeros_hbm, acc.at[pl.ds(s * _PER_SUB, _PER_SUB)])

        @pl.when(s < _NS - 1)
        def _():
            pltpu.sync_copy(
                fsplit_hbm.at[c].at[pl.ds(s * _PER_SUB, _PER_SUB)],
                feat_s.at[pl.ds(s * _PER_SUB, _PER_SUB)])

        @pl.when(s == _NS - 1)
        def _():
            pltpu.sync_copy(
                fsplit_hbm.at[c].at[pl.ds((_NS - 1) * _PER_SUB, _TAIL)],
                feat_s.at[pl.ds((_NS - 1) * _PER_SUB, _TAIL)])

        plsc.subcore_barrier()

        def convert(rb, rf):
            def crow(r, carry):
                a, b = plsc.unpack(rb.at[r][...],
                                   format=plsc.PackFormat.INTERLEAVED)
                rf.at[r][pl.ds(0, 16)] = a
                rf.at[r][pl.ds(16, 16)] = b
                return carry

            lax.fori_loop(0, _CHUNK, crow, 0)

        def body(i, carry):
            j0 = i * 2
            j1 = j0 + 1
            g0 = pltpu.async_copy(feat_s.at[sidx.at[j0]], rb0, gsem0)
            g1 = pltpu.async_copy(feat_s.at[sidx.at[j1]], rb1, gsem1)
            g0.wait()
            convert(rb0, rows0)
            s0 = pltpu.async_copy(rows0, acc.at[didx.at[j0]], ssem0,
                                  add=True)
            g1.wait()
            convert(rb1, rows1)
            s1 = pltpu.async_copy(rows1, acc.at[didx.at[j1]], ssem1,
                                  add=True)
            s0.wait()
            s1.wait()
            return carry

        lax.fori_loop(0, _BLK // 2, body, 0)
        plsc.subcore_barrier()

        @pl.when(c == 0)
        def _():
            pltpu.sync_copy(acc.at[pl.ds(s * _PER_SUB, _PER_SUB)],
                            out0_hbm.at[pl.ds(s * _PER_SUB, _PER_SUB)])

        @pl.when(c == 1)
        def _():
            pltpu.sync_copy(acc.at[pl.ds(s * _PER_SUB, _PER_SUB)],
                            out1_hbm.at[pl.ds(s * _PER_SUB, _PER_SUB)])

    return _sc_segment_sum


def kernel(x_radiomics, x_pathomics, W_er, b_er, W_ep, b_ep, W_h, b_h,
           W_g, b_g, Wa, ba, Wb, bb, Wc, bc, W_t, b_t, edge_index):
    n, h = _N, _H

    feat = pl.pallas_call(
        _encoder_body,
        out_shape=jax.ShapeDtypeStruct((n, h), jnp.float32),
    )(x_radiomics, x_pathomics, W_er, W_ep, W_h)

    fb = feat.astype(jnp.bfloat16).reshape(n, _NC, 2, _HC // 2)
    fsplit = jnp.transpose(fb, (1, 0, 3, 2)).reshape(_NC, n, _HC)

    src = edge_index[0]
    dst = edge_index[1]
    e = src.shape[0]
    pad = _EPAD - e
    pad_src = (jnp.arange(pad, dtype=jnp.int32) * 197) % n
    pad_dst = n + (jnp.arange(pad, dtype=jnp.int32) % (_ACC_ROWS - n))
    src_p = jnp.concatenate([src, pad_src]).reshape(_NS, _BLK, _CHUNK)
    dst_p = jnp.concatenate([dst, pad_dst]).reshape(_NS, _BLK, _CHUNK)
    zeros = jnp.zeros((_PER_SUB, _HC), jnp.float32)

    part0, part1 = _get_sc_segment_sum()(fsplit, src_p, dst_p, zeros)

    out = pl.pallas_call(
        _head_body,
        out_shape=jax.ShapeDtypeStruct((1, 1), jnp.float32),
    )(feat, part0, part1, W_g, Wa, ba.reshape(1, -1), Wb, bb.reshape(1, -1),
      Wc, W_t, b_t.reshape(1, 1))
    return out


# speedup vs baseline: 1.5446x; 1.4306x over previous
import functools

import jax
import jax.numpy as jnp
from jax import lax
from jax.experimental import pallas as pl
from jax.experimental.pallas import tpu as pltpu
from jax.experimental.pallas import tpu_sc as plsc

_N = 10000
_H = 64
_HC = _H // 2

_NC = 2
_NS = 16
_CHUNK = 512
_BLK = 40
_EPW = _BLK * _CHUNK
_EPAD = _NS * _EPW
_ACC_ROWS = 10112
_PER_SUB = _ACC_ROWS // _NS
_TAIL = _N - (_NS - 1) * _PER_SUB


def _bn_relu(y):
    mu = jnp.mean(y, axis=0, keepdims=True)
    yc = y - mu
    var = jnp.mean(yc * yc, axis=0, keepdims=True)
    return jnp.maximum(yc * lax.rsqrt(var + 1e-5), 0.0)


def _encoder_body(xr_ref, xp_ref, wer_ref, wep_ref, wh_ref,
                  feat_ref, fsplit_ref):
    hr = _bn_relu(jnp.dot(xr_ref[...], wer_ref[...],
                          preferred_element_type=jnp.float32))
    hp = _bn_relu(jnp.dot(xp_ref[...], wep_ref[...],
                          preferred_element_type=jnp.float32))
    f = hr + hp
    feat = _bn_relu(jnp.dot(f, wh_ref[...],
                            preferred_element_type=jnp.float32))
    feat_ref[...] = feat
    fsplit_ref[0] = feat[:, :_HC]
    fsplit_ref[1] = feat[:, _HC:]


def _head_body(feat_ref, p0_ref, p1_ref, wg_ref, wa_ref, ba_ref, wb_ref,
               bb_ref, wc_ref, wt_ref, bt_ref, out_ref):
    f = feat_ref[...]
    agg = jnp.concatenate([p0_ref[:_N, :], p1_ref[:_N, :]], axis=1)
    g = _bn_relu(jnp.dot(f + agg, wg_ref[...],
                         preferred_element_type=jnp.float32))
    a = jnp.tanh(jnp.dot(g, wa_ref[...],
                         preferred_element_type=jnp.float32) + ba_ref[...])
    b = jax.nn.sigmoid(jnp.dot(g, wb_ref[...],
                               preferred_element_type=jnp.float32)
                       + bb_ref[...])
    s = jnp.dot(a * b, wc_ref[...],
                preferred_element_type=jnp.float32)
    m = jnp.max(s)
    e = jnp.exp(s - m)
    z = jnp.sum(e)
    pooled = jnp.sum(e * g, axis=0, keepdims=True) / z
    out_ref[...] = jnp.dot(pooled, wt_ref[...],
                           preferred_element_type=jnp.float32) + bt_ref[...]


@functools.lru_cache(maxsize=1)
def _get_sc_segment_sum():
    mesh = plsc.VectorSubcoreMesh(core_axis_name="c", subcore_axis_name="s")

    @functools.partial(
        pl.kernel,
        out_type=(jax.ShapeDtypeStruct((_ACC_ROWS, _HC), jnp.float32),
                  jax.ShapeDtypeStruct((_ACC_ROWS, _HC), jnp.float32)),
        mesh=mesh,
        compiler_params=pltpu.CompilerParams(use_tc_tiling_on_sc=False),
        scratch_types=[
            pltpu.VMEM((_BLK, _CHUNK), jnp.int32),
            pltpu.VMEM((_BLK, _CHUNK), jnp.int32),
            pltpu.VMEM((_CHUNK, _HC), jnp.float32),
            pltpu.VMEM((_CHUNK, _HC), jnp.float32),
            pltpu.VMEM_SHARED((_ACC_ROWS, _HC), jnp.float32),
            pltpu.VMEM_SHARED((_N, _HC), jnp.float32),
            pltpu.SemaphoreType.DMA,
            pltpu.SemaphoreType.DMA,
            pltpu.SemaphoreType.DMA,
            pltpu.SemaphoreType.DMA,
        ],
    )
    def _sc_segment_sum(fsplit_hbm, src_hbm, dst_hbm, zeros_hbm,
                        out0_hbm, out1_hbm,
                        sidx, didx, rows0, rows1, acc, feat_s,
                        gsem0, gsem1, ssem0, ssem1):
        c = lax.axis_index("c")
        s = lax.axis_index("s")
        pltpu.sync_copy(src_hbm.at[s], sidx)
        pltpu.sync_copy(dst_hbm.at[s], didx)
        pltpu.sync_copy(zeros_hbm, acc.at[pl.ds(s * _PER_SUB, _PER_SUB)])

        @pl.when(s < _NS - 1)
        def _():
            pltpu.sync_copy(
                fsplit_hbm.at[c].at[pl.ds(s * _PER_SUB, _PER_SUB)],
                feat_s.at[pl.ds(s * _PER_SUB, _PER_SUB)])

        @pl.when(s == _NS - 1)
        def _():
            pltpu.sync_copy(
                fsplit_hbm.at[c].at[pl.ds((_NS - 1) * _PER_SUB, _TAIL)],
                feat_s.at[pl.ds((_NS - 1) * _PER_SUB, _TAIL)])

        plsc.subcore_barrier()

        def body(i, carry):
            j0 = i * 2
            j1 = j0 + 1
            g0 = pltpu.async_copy(feat_s.at[sidx.at[j0]], rows0, gsem0)
            g1 = pltpu.async_copy(feat_s.at[sidx.at[j1]], rows1, gsem1)
            g0.wait()
            s0 = pltpu.async_copy(rows0, acc.at[didx.at[j0]], ssem0,
                                  add=True)
            g1.wait()
            s1 = pltpu.async_copy(rows1, acc.at[didx.at[j1]], ssem1,
                                  add=True)
            s0.wait()
            s1.wait()
            return carry

        lax.fori_loop(0, _BLK // 2, body, 0)
        plsc.subcore_barrier()

        @pl.when(c == 0)
        def _():
            pltpu.sync_copy(acc.at[pl.ds(s * _PER_SUB, _PER_SUB)],
                            out0_hbm.at[pl.ds(s * _PER_SUB, _PER_SUB)])

        @pl.when(c == 1)
        def _():
            pltpu.sync_copy(acc.at[pl.ds(s * _PER_SUB, _PER_SUB)],
                            out1_hbm.at[pl.ds(s * _PER_SUB, _PER_SUB)])

    return _sc_segment_sum


def kernel(x_radiomics, x_pathomics, W_er, b_er, W_ep, b_ep, W_h, b_h,
           W_g, b_g, Wa, ba, Wb, bb, Wc, bc, W_t, b_t, edge_index):
    n, h = _N, _H

    feat, fsplit = pl.pallas_call(
        _encoder_body,
        out_shape=(jax.ShapeDtypeStruct((n, h), jnp.float32),
                   jax.ShapeDtypeStruct((_NC, n, _HC), jnp.float32)),
    )(x_radiomics, x_pathomics, W_er, W_ep, W_h)

    src = edge_index[0]
    dst = edge_index[1]
    e = src.shape[0]
    pad = _EPAD - e
    pad_src = (jnp.arange(pad, dtype=jnp.int32) * 197) % n
    pad_dst = n + (jnp.arange(pad, dtype=jnp.int32) % (_ACC_ROWS - n))
    src_p = jnp.concatenate([src, pad_src]).reshape(_NS, _BLK, _CHUNK)
    dst_p = jnp.concatenate([dst, pad_dst]).reshape(_NS, _BLK, _CHUNK)
    zeros = jnp.zeros((_PER_SUB, _HC), jnp.float32)

    part0, part1 = _get_sc_segment_sum()(fsplit, src_p, dst_p, zeros)

    out = pl.pallas_call(
        _head_body,
        out_shape=jax.ShapeDtypeStruct((1, 1), jnp.float32),
    )(feat, part0, part1, W_g, Wa, ba.reshape(1, -1), Wb, bb.reshape(1, -1),
      Wc, W_t, b_t.reshape(1, 1))
    return out
